# baseline (speedup 1.0000x reference)
import functools

import jax
import jax.numpy as jnp
from jax import lax
from jax.experimental import pallas as pl
from jax.experimental.pallas import tpu as pltpu
from jax.experimental.pallas import tpu_sc as plsc

_N_EDGES = 320000
_EMB = 128
_VOCAB = 5

_NC = 2
_NS = 16
_NW = _NC * _NS
_B_PER_W = _N_EDGES // _NS
_CHUNK = 400
_NCHUNK = _B_PER_W // _CHUNK
_GROUPS = _CHUNK // 16
_NPAIR = _NCHUNK // 2


@functools.partial(
    pl.kernel,
    out_type=jax.ShapeDtypeStruct((_N_EDGES * _EMB,), jnp.float32),
    mesh=plsc.VectorSubcoreMesh(core_axis_name="c", subcore_axis_name="s"),
    compiler_params=pltpu.CompilerParams(
        needs_layout_passes=False, disable_bounds_checks=True),
    scratch_types=[
        pltpu.VMEM((_CHUNK * 3,), jnp.int32),
        pltpu.VMEM((_CHUNK * 3,), jnp.int32),
        pltpu.VMEM((_CHUNK * _EMB,), jnp.float32),
        pltpu.VMEM((_CHUNK * _EMB,), jnp.float32),
        pltpu.VMEM((_VOCAB * _EMB,), jnp.float32),
        pltpu.SemaphoreType.DMA,
        pltpu.SemaphoreType.DMA,
        pltpu.SemaphoreType.DMA,
        pltpu.SemaphoreType.DMA,
    ],
)
def _embed_gather(attr_hbm, table_hbm, out_hbm, attr_v0, attr_v1,
                  out_v0, out_v1, table_v,
                  sem_in0, sem_in1, sem_out0, sem_out1):
    attr_bufs = (attr_v0, attr_v1)
    out_bufs = (out_v0, out_v1)
    sems_in = (sem_in0, sem_in1)
    sems_out = (sem_out0, sem_out1)
    core = lax.axis_index("c")
    base = lax.axis_index("s") * _B_PER_W
    lanes = lax.iota(jnp.int32, 16)

    def start_in(c, b):
        return pltpu.async_copy(
            attr_hbm.at[pl.ds((base + c * _CHUNK) * 3, _CHUNK * 3)],
            attr_bufs[b], sems_in[b])

    def wait_in(b):
        pltpu.make_async_copy(
            attr_hbm.at[pl.ds(0, _CHUNK * 3)], attr_bufs[b], sems_in[b]
        ).wait()

    def start_out(c, b):
        return pltpu.async_copy(
            out_bufs[b],
            out_hbm.at[pl.ds((base + c * _CHUNK) * _EMB, _CHUNK * _EMB)],
            sems_out[b])

    def wait_out(b):
        pltpu.make_async_copy(
            out_bufs[b], out_hbm.at[pl.ds(0, _CHUNK * _EMB)], sems_out[b]
        ).wait()

    def compute(b):
        av = attr_bufs[b]
        ov = out_bufs[b]

        @plsc.parallel_loop(0, _GROUPS, unroll=2)
        def group_body(g):
            srcs = plsc.load_gather(av, [(g * 16 + lanes) * 3]) * _EMB
            obase = g * (16 * _EMB)
            for e in range(16):
                src = srcs[e]
                dst = obase + e * _EMB
                for j in range(0, _EMB, 16):
                    ov[pl.ds(dst + j, 16)] = table_v[pl.ds(src + j, 16)]

    @pl.when(core == 0)
    def _all_work():
        pltpu.sync_copy(table_hbm, table_v)
        start_in(0, 0)
        start_in(1, 1)

        def pair_body(p, carry):
            c0 = p * 2
            for b in range(2):
                c = c0 + b
                wait_in(b)

                @pl.when(p > 0)
                def _():
                    wait_out(b)

                compute(b)
                start_out(c, b)

                @pl.when(c + 2 < _NCHUNK)
                def _():
                    start_in(c + 2, b)

            return carry

        lax.fori_loop(0, _NPAIR, pair_body, 0)

        wait_out(0)
        wait_out(1)


def kernel(edge_attr, bond_embedding):
    out = _embed_gather(edge_attr.reshape(-1), bond_embedding.reshape(-1))
    return out.reshape(_N_EDGES, _EMB)


# speedup vs baseline: 1.3151x; 1.3151x over previous
import functools

import jax
import jax.numpy as jnp
from jax import lax
from jax.experimental import pallas as pl
from jax.experimental.pallas import tpu as pltpu
from jax.experimental.pallas import tpu_sc as plsc

_N_EDGES = 320000
_EMB = 128
_VOCAB = 5

_NC = 2
_NS = 16
_NW = _NC * _NS
_B_PER_W = _N_EDGES // _NW
_CHUNK = 400
_NCHUNK = _B_PER_W // _CHUNK
_GROUPS = _CHUNK // 16
_NPAIR = _NCHUNK // 2


@functools.partial(
    pl.kernel,
    out_type=jax.ShapeDtypeStruct((_N_EDGES * _EMB,), jnp.float32),
    mesh=plsc.VectorSubcoreMesh(core_axis_name="c", subcore_axis_name="s"),
    compiler_params=pltpu.CompilerParams(
        needs_layout_passes=False, disable_bounds_checks=True),
    scratch_types=[
        pltpu.VMEM((_CHUNK * 3,), jnp.int32),
        pltpu.VMEM((_CHUNK * 3,), jnp.int32),
        pltpu.VMEM((_CHUNK * _EMB,), jnp.float32),
        pltpu.VMEM((_CHUNK * _EMB,), jnp.float32),
        pltpu.VMEM((_VOCAB * _EMB,), jnp.float32),
        pltpu.SemaphoreType.DMA,
        pltpu.SemaphoreType.DMA,
        pltpu.SemaphoreType.DMA,
        pltpu.SemaphoreType.DMA,
    ],
)
def _embed_gather(attr_hbm, table_hbm, out_hbm, attr_v0, attr_v1,
                  out_v0, out_v1, table_v,
                  sem_in0, sem_in1, sem_out0, sem_out1):
    attr_bufs = (attr_v0, attr_v1)
    out_bufs = (out_v0, out_v1)
    sems_in = (sem_in0, sem_in1)
    sems_out = (sem_out0, sem_out1)
    wid = lax.axis_index("s") * _NC + lax.axis_index("c")
    base = wid * _B_PER_W
    pltpu.sync_copy(table_hbm, table_v)
    lanes = lax.iota(jnp.int32, 16)

    def start_in(c, b):
        return pltpu.async_copy(
            attr_hbm.at[pl.ds((base + c * _CHUNK) * 3, _CHUNK * 3)],
            attr_bufs[b], sems_in[b])

    def wait_in(b):
        pltpu.make_async_copy(
            attr_hbm.at[pl.ds(0, _CHUNK * 3)], attr_bufs[b], sems_in[b]
        ).wait()

    def start_out(c, b):
        return pltpu.async_copy(
            out_bufs[b],
            out_hbm.at[pl.ds((base + c * _CHUNK) * _EMB, _CHUNK * _EMB)],
            sems_out[b])

    def wait_out(b):
        pltpu.make_async_copy(
            out_bufs[b], out_hbm.at[pl.ds(0, _CHUNK * _EMB)], sems_out[b]
        ).wait()

    def compute(b):
        av = attr_bufs[b]
        ov = out_bufs[b]

        @plsc.parallel_loop(0, _GROUPS, unroll=2)
        def group_body(g):
            srcs = plsc.load_gather(av, [(g * 16 + lanes) * 3]) * _EMB
            obase = g * (16 * _EMB)
            for e in range(16):
                src = srcs[e]
                dst = obase + e * _EMB
                for j in range(0, _EMB, 16):
                    ov[pl.ds(dst + j, 16)] = table_v[pl.ds(src + j, 16)]

    start_in(0, 0)
    start_in(1, 1)

    def pair_body(p, carry):
        c0 = p * 2
        for b in range(2):
            c = c0 + b
            wait_in(b)

            @pl.when(p > 0)
            def _():
                wait_out(b)

            compute(b)
            start_out(c, b)

            @pl.when(c + 2 < _NCHUNK)
            def _():
                start_in(c + 2, b)

        return carry

    lax.fori_loop(0, _NPAIR, pair_body, 0)

    wait_in(0)
    wait_out(0)
    compute(0)
    start_out(_NCHUNK - 1, 0)
    wait_out(0)
    wait_out(1)


def kernel(edge_attr, bond_embedding):
    out = _embed_gather(edge_attr.reshape(-1), bond_embedding.reshape(-1))
    return out.reshape(_N_EDGES, _EMB)
